# baseline (speedup 1.0000x reference)
import jax
import jax.numpy as jnp
from jax.experimental import pallas as pl

N = 10000
HEADS = 8
HIDDEN = 128
OUT_DIM = 128


def _mm_kernel(x_ref, w_ref, o_ref):
    o_ref[...] = jnp.dot(x_ref[...], w_ref[...],
                         preferred_element_type=jnp.float32)


def _matmul(x, w, bm=256):
    m, k = x.shape
    k2, n = w.shape
    pad = (-m) % bm
    xp = jnp.pad(x, ((0, pad), (0, 0)))
    out = pl.pallas_call(
        _mm_kernel,
        grid=((m + pad) // bm,),
        in_specs=[pl.BlockSpec((bm, k), lambda i: (i, 0)),
                  pl.BlockSpec((k2, n), lambda i: (0, 0))],
        out_specs=pl.BlockSpec((bm, n), lambda i: (i, 0)),
        out_shape=jax.ShapeDtypeStruct((m + pad, n), jnp.float32),
    )(xp, w)
    return out[:m]


def _leaky(v):
    return jnp.where(v >= 0, v, 0.2 * v)


def _gat(x, src, dst, W, att_s, att_d, bias, heads, out_ch, concat):
    n = x.shape[0]
    h = _matmul(x, W).reshape(n, heads, out_ch)
    a_s = (h * att_s[None, :, :]).sum(-1)
    a_d = (h * att_d[None, :, :]).sum(-1)
    alpha = _leaky(a_s[src] + a_d[dst])
    amax = jax.ops.segment_max(alpha, dst, num_segments=n)
    ex = jnp.exp(alpha - amax[dst])
    denom = jax.ops.segment_sum(ex, dst, num_segments=n)
    coef = ex / (denom[dst] + 1e-16)
    out = jax.ops.segment_sum(h[src] * coef[:, :, None], dst, num_segments=n)
    if concat:
        return out.reshape(n, heads * out_ch) + bias
    return out.mean(axis=1) + bias


def kernel(x, edge_index, edge_attr, We, be, W1, as1, ad1, b1, W2, as2, ad2, b2):
    n = x.shape[0]
    src0, dst0 = edge_index[0], edge_index[1]
    ee = _matmul(edge_attr, We) + be
    s = jax.ops.segment_sum(ee, dst0, num_segments=n)
    cnt = jax.ops.segment_sum(jnp.ones((ee.shape[0],), x.dtype), dst0,
                              num_segments=n)
    x = x + s / jnp.maximum(cnt, 1.0)[:, None]
    loop = jnp.arange(n, dtype=src0.dtype)
    src = jnp.concatenate([src0, loop])
    dst = jnp.concatenate([dst0, loop])
    x = jax.nn.relu(_gat(x, src, dst, W1, as1, ad1, b1, HEADS, HIDDEN, True))
    x = _gat(x, src, dst, W2, as2, ad2, b2, 1, OUT_DIM, False)
    return x.mean(axis=0)


# speedup vs baseline: 12.3331x; 12.3331x over previous
import jax
import jax.numpy as jnp
from jax import lax
from jax.experimental import pallas as pl
from jax.experimental.pallas import tpu as pltpu
from jax.experimental.pallas import tpu_sc as plsc

N = 10000
E = 320000
ND = 128
HEADS = 8
HID = 128
OUT_DIM = 128
CAP = 16.0

NC = 2
NT = 16
L = 16

K = 80

EPT = E // (NC * NT)
NFULL = EPT // K

EPS = E // NT
NF4 = EPS // K

_MESH = plsc.VectorSubcoreMesh(core_axis_name="c", subcore_axis_name="s")

_f32 = jnp.float32
_i32 = jnp.int32


def _leaky(v):
    return jnp.where(v >= 0, v, 0.2 * v)


SLAB = 624
REM_BASE = SLAB * NT
REM = N - REM_BASE


def _tile_copy(src, dst, s):
    pltpu.sync_copy(src.at[pl.ds(s * SLAB, SLAB)],
                    dst.at[pl.ds(s * SLAB, SLAB)])

    @pl.when(s == NT - 1)
    def _():
        pltpu.sync_copy(src.at[pl.ds(REM_BASE, REM)],
                        dst.at[pl.ds(REM_BASE, REM)])


def _tile_readout(accum, out_slice, s):
    pltpu.sync_copy(accum.at[pl.ds(s * SLAB, SLAB)],
                    out_slice.at[pl.ds(s * SLAB, SLAB)])

    @pl.when(s == NT - 1)
    def _():
        pltpu.sync_copy(accum.at[pl.ds(REM_BASE, REM)],
                        out_slice.at[pl.ds(REM_BASE, REM)])


def _s1_body(edst, eattr, z32, out, accum, attrbuf, rowbuf, didx):
    c = lax.axis_index("c")
    s = lax.axis_index("s")
    tbase = (c * NT + s) * EPT
    _tile_copy(z32, accum, s)
    plsc.subcore_barrier()
    vone = jnp.where(lax.iota(_i32, L) == 0,
                     jnp.full((L,), 1.0, _f32), jnp.full((L,), 0.0, _f32))

    def body(i, carry):
        base = tbase + i * K
        pltpu.sync_copy(eattr.at[pl.ds(base, K)], attrbuf)
        pltpu.sync_copy(edst.at[pl.ds(base, K)], didx)
        for kk in range(K):
            rowbuf[kk, 0:16] = attrbuf[kk, :]
            rowbuf[kk, 16:32] = vone
        pltpu.sync_copy(rowbuf, accum.at[didx], add=True)
        return carry

    lax.fori_loop(0, NFULL, body, 0)
    plsc.subcore_barrier()
    _tile_readout(accum, out.at[c], s)


def _s1_call(edst, edge_attr):
    z32 = jnp.zeros((N, 32), _f32)
    return pl.kernel(
        _s1_body,
        out_type=jax.ShapeDtypeStruct((NC, N, 32), _f32),
        mesh=_MESH,
        scratch_types=[
            pltpu.VMEM_SHARED((N, 32), _f32),
            pltpu.VMEM((K, 16), _f32),
            pltpu.VMEM((K, 32), _f32),
            pltpu.VMEM((K,), _i32),
        ],
        compiler_params=pltpu.CompilerParams(use_tc_tiling_on_sc=False),
    )(edst, edge_attr, z32)


def _s2_body(esrc, edst, tabs, tabd, tabec, z16, ex_out, den_out,
             accum, va, vb, vc, exbuf, sidx, didx, sem):
    c = lax.axis_index("c")
    s = lax.axis_index("s")
    tbase = (c * NT + s) * EPT
    _tile_copy(z16, accum, s)
    plsc.subcore_barrier()

    def body(i, carry):
        base = tbase + i * K
        pltpu.sync_copy(esrc.at[pl.ds(base, K)], sidx)
        pltpu.sync_copy(edst.at[pl.ds(base, K)], didx)
        d1 = pltpu.async_copy(tabs.at[sidx], va, sem)
        d2 = pltpu.async_copy(tabd.at[didx], vb, sem)
        d3 = pltpu.async_copy(tabec.at[didx], vc, sem)
        d1.wait()
        d2.wait()
        d3.wait()
        for kk in range(K):
            v = va[kk, :] + vb[kk, :]
            v = jnp.exp(_leaky(v)) * vc[kk, :]
            exbuf[kk, :] = v
        pltpu.sync_copy(exbuf, accum.at[didx], add=True)
        pltpu.sync_copy(exbuf, ex_out.at[pl.ds(base, K)])
        return carry

    lax.fori_loop(0, NFULL, body, 0)
    plsc.subcore_barrier()
    _tile_readout(accum, den_out.at[c], s)


def _s2_call(esrc, edst, tabs, tabd, tabec):
    z16 = jnp.zeros((N, 16), _f32)
    return pl.kernel(
        _s2_body,
        out_type=(jax.ShapeDtypeStruct((E, 16), _f32),
                  jax.ShapeDtypeStruct((NC, N, 16), _f32)),
        mesh=_MESH,
        scratch_types=[
            pltpu.VMEM_SHARED((N, 16), _f32),
            pltpu.VMEM((K, 16), _f32),
            pltpu.VMEM((K, 16), _f32),
            pltpu.VMEM((K, 16), _f32),
            pltpu.VMEM((K, 16), _f32),
            pltpu.VMEM((K,), _i32),
            pltpu.VMEM((K,), _i32),
            pltpu.SemaphoreType.DMA,
        ],
        compiler_params=pltpu.CompilerParams(use_tc_tiling_on_sc=False),
    )(esrc, edst, tabs, tabd, tabec, z16)


def _s4_body(esrc, edst, ex1, h1flat, z128, out, accum,
             hbuf, exb, sidx, didx, aidx, sem):
    c = lax.axis_index("c")
    s = lax.axis_index("s")
    tbase = s * EPS

    def head_body(hh, hcarry):
        hd = c * 4 + hh
        _tile_copy(z128, accum, s)
        plsc.subcore_barrier()
        hdv = jnp.full((L,), hd, _i32)
        off = hd * N

        def body(i, carry):
            base = tbase + i * K
            pltpu.sync_copy(esrc.at[pl.ds(base, K)], sidx)
            pltpu.sync_copy(edst.at[pl.ds(base, K)], didx)
            for j in range(K // L):
                aidx[pl.ds(j * L, L)] = sidx[pl.ds(j * L, L)] + off
            d1 = pltpu.async_copy(h1flat.at[aidx], hbuf, sem)
            pltpu.sync_copy(ex1.at[pl.ds(base, K)], exb)
            d1.wait()
            for kk in range(K):
                sc = plsc.load_gather(exb, [jnp.full((L,), kk, _i32), hdv])
                for j in range(8):
                    hbuf[kk, j * L:(j + 1) * L] = (
                        hbuf[kk, j * L:(j + 1) * L] * sc)
            pltpu.sync_copy(hbuf, accum.at[didx], add=True)
            return carry

        lax.fori_loop(0, NF4, body, 0)
        plsc.subcore_barrier()
        _tile_readout(accum, out.at[hd], s)
        plsc.subcore_barrier()
        return hcarry

    lax.fori_loop(0, 4, head_body, 0)


def _s4_call(esrc, edst, ex1, h1flat):
    z128 = jnp.zeros((N, 128), _f32)
    return pl.kernel(
        _s4_body,
        out_type=jax.ShapeDtypeStruct((HEADS, N, 128), _f32),
        mesh=_MESH,
        scratch_types=[
            pltpu.VMEM_SHARED((N, 128), _f32),
            pltpu.VMEM((K, 128), _f32),
            pltpu.VMEM((K, 16), _f32),
            pltpu.VMEM((K,), _i32),
            pltpu.VMEM((K,), _i32),
            pltpu.VMEM((K,), _i32),
            pltpu.SemaphoreType.DMA,
        ],
        compiler_params=pltpu.CompilerParams(needs_layout_passes=False,
                                             use_tc_tiling_on_sc=False),
    )(esrc, edst, ex1, h1flat, z128)


def _s5a_body(esrc, edst, as2v, ad2v, ec2v, z16, den_out,
              denacc, as2t, ad2t, ec2t, exrows, sidx, didx):
    c = lax.axis_index("c")
    s = lax.axis_index("s")
    tbase = (c * NT + s) * EPT
    _tile_copy(z16, denacc, s)
    pltpu.sync_copy(as2v, as2t)
    pltpu.sync_copy(ad2v, ad2t)
    pltpu.sync_copy(ec2v, ec2t)
    pltpu.sync_copy(z16.at[pl.ds(0, K)], exrows)
    plsc.subcore_barrier()
    rowi = lax.iota(_i32, L)
    zi = jnp.zeros((L,), _i32)

    def body(i, carry):
        base = tbase + i * K
        pltpu.sync_copy(esrc.at[pl.ds(base, K)], sidx)
        pltpu.sync_copy(edst.at[pl.ds(base, K)], didx)
        for j in range(K // L):
            sv = sidx[pl.ds(j * L, L)]
            dv = didx[pl.ds(j * L, L)]
            va = plsc.load_gather(as2t, [sv])
            vb = plsc.load_gather(ad2t, [dv])
            vc = plsc.load_gather(ec2t, [dv])
            v = va + vb
            ex = jnp.exp(0.6 * v + 0.4 * jnp.abs(v)) * vc
            plsc.store_scatter(exrows, [rowi + j * L, zi], ex)
        pltpu.sync_copy(exrows, denacc.at[didx], add=True)
        return carry

    lax.fori_loop(0, NFULL, body, 0)
    plsc.subcore_barrier()
    _tile_readout(denacc, den_out.at[c], s)


def _s5a_call(esrc, edst, as2v, ad2v, ec2v):
    z16 = jnp.zeros((N, 16), _f32)
    return pl.kernel(
        _s5a_body,
        out_type=jax.ShapeDtypeStruct((NC, N, 16), _f32),
        mesh=_MESH,
        scratch_types=[
            pltpu.VMEM_SHARED((N, 16), _f32),
            pltpu.VMEM((N,), _f32),
            pltpu.VMEM((N,), _f32),
            pltpu.VMEM((N,), _f32),
            pltpu.VMEM((K, 16), _f32),
            pltpu.VMEM((K,), _i32),
            pltpu.VMEM((K,), _i32),
        ],
        compiler_params=pltpu.CompilerParams(needs_layout_passes=False,
                                             use_tc_tiling_on_sc=False),
    )(esrc, edst, as2v, ad2v, ec2v, z16)


def _s5b_body(esrc, edst, as2v, ad2v, ec2v, h2, z128, out,
              accum, as2t, ad2t, ec2t, hbuf, exvec, sidx, didx, sem):
    c = lax.axis_index("c")
    s = lax.axis_index("s")
    tbase = (c * NT + s) * EPT
    _tile_copy(z128, accum, s)
    pltpu.sync_copy(as2v, as2t)
    pltpu.sync_copy(ad2v, ad2t)
    pltpu.sync_copy(ec2v, ec2t)
    plsc.subcore_barrier()

    def body(i, carry):
        base = tbase + i * K
        pltpu.sync_copy(esrc.at[pl.ds(base, K)], sidx)
        pltpu.sync_copy(edst.at[pl.ds(base, K)], didx)
        d1 = pltpu.async_copy(h2.at[sidx], hbuf, sem)
        for j in range(K // L):
            sv = sidx[pl.ds(j * L, L)]
            dv = didx[pl.ds(j * L, L)]
            va = plsc.load_gather(as2t, [sv])
            vb = plsc.load_gather(ad2t, [dv])
            vc = plsc.load_gather(ec2t, [dv])
            v = va + vb
            ex = jnp.exp(0.6 * v + 0.4 * jnp.abs(v)) * vc
            exvec[pl.ds(j * L, L)] = ex
        d1.wait()
        for kk in range(K):
            sc = plsc.load_gather(exvec, [jnp.full((L,), kk, _i32)])
            for j in range(8):
                hbuf[kk, j * L:(j + 1) * L] = hbuf[kk, j * L:(j + 1) * L] * sc
        pltpu.sync_copy(hbuf, accum.at[didx], add=True)
        return carry

    lax.fori_loop(0, NFULL, body, 0)
    plsc.subcore_barrier()
    _tile_readout(accum, out.at[c], s)


def _s5b_call(esrc, edst, as2v, ad2v, ec2v, h2):
    z128 = jnp.zeros((N, 128), _f32)
    return pl.kernel(
        _s5b_body,
        out_type=jax.ShapeDtypeStruct((NC, N, 128), _f32),
        mesh=_MESH,
        scratch_types=[
            pltpu.VMEM_SHARED((N, 128), _f32),
            pltpu.VMEM((N,), _f32),
            pltpu.VMEM((N,), _f32),
            pltpu.VMEM((N,), _f32),
            pltpu.VMEM((K, 128), _f32),
            pltpu.VMEM((K,), _f32),
            pltpu.VMEM((K,), _i32),
            pltpu.VMEM((K,), _i32),
            pltpu.SemaphoreType.DMA,
        ],
        compiler_params=pltpu.CompilerParams(needs_layout_passes=False,
                                             use_tc_tiling_on_sc=False),
    )(esrc, edst, as2v, ad2v, ec2v, h2, z128)


BM = 1000


def _t1_body(x_ref, s32_ref, we_ref, be_ref, w1_ref, as1_ref, ad1_ref,
             h1tab_ref, tabs_ref, tabd_ref, tabec_ref, selfex_ref):
    S = s32_ref[0] + s32_ref[1]
    cnt = S[:, 16:17]
    attrm = S[:, 0:16] / jnp.maximum(cnt, 1.0)
    etn = jnp.dot(attrm, we_ref[...], preferred_element_type=_f32)
    etn = etn + be_ref[...] * jnp.minimum(cnt, 1.0)
    xp = x_ref[...] + etn
    a_s_cols = []
    a_d_cols = []
    for hd in range(HEADS):
        h = jnp.dot(xp, w1_ref[:, hd * HID:(hd + 1) * HID],
                    preferred_element_type=_f32)
        h1tab_ref[hd] = h
        a_s_cols.append((h * as1_ref[hd][None, :]).sum(-1, keepdims=True))
        a_d_cols.append((h * ad1_ref[hd][None, :]).sum(-1, keepdims=True))
    a_s = jnp.concatenate(a_s_cols, axis=1)
    a_d = jnp.concatenate(a_d_cols, axis=1)
    ec = jnp.exp(-_leaky(CAP + a_d))
    selfex = jnp.exp(_leaky(a_s + a_d)) * ec
    pad = jnp.zeros((BM, 8), _f32)
    tabs_ref[...] = jnp.concatenate([a_s, pad], axis=1)
    tabd_ref[...] = jnp.concatenate([a_d, pad], axis=1)
    tabec_ref[...] = jnp.concatenate([ec, pad], axis=1)
    selfex_ref[...] = jnp.concatenate([selfex, pad], axis=1)


def _t1_call(x, s32, We, be, W1, as1, ad1):
    grid = N // BM
    return pl.pallas_call(
        _t1_body,
        grid=(grid,),
        in_specs=[
            pl.BlockSpec((BM, ND), lambda i: (i, 0)),
            pl.BlockSpec((2, BM, 32), lambda i: (0, i, 0)),
            pl.BlockSpec((16, ND), lambda i: (0, 0)),
            pl.BlockSpec((1, ND), lambda i: (0, 0)),
            pl.BlockSpec((ND, HEADS * HID), lambda i: (0, 0)),
            pl.BlockSpec((HEADS, HID), lambda i: (0, 0)),
            pl.BlockSpec((HEADS, HID), lambda i: (0, 0)),
        ],
        out_specs=[
            pl.BlockSpec((HEADS, BM, HID), lambda i: (0, i, 0)),
            pl.BlockSpec((BM, 16), lambda i: (i, 0)),
            pl.BlockSpec((BM, 16), lambda i: (i, 0)),
            pl.BlockSpec((BM, 16), lambda i: (i, 0)),
            pl.BlockSpec((BM, 16), lambda i: (i, 0)),
        ],
        out_shape=[
            jax.ShapeDtypeStruct((HEADS, N, HID), _f32),
            jax.ShapeDtypeStruct((N, 16), _f32),
            jax.ShapeDtypeStruct((N, 16), _f32),
            jax.ShapeDtypeStruct((N, 16), _f32),
            jax.ShapeDtypeStruct((N, 16), _f32),
        ],
    )(x, s32, We, be.reshape(1, ND), W1, as1, ad1)


def _t6_body(scat1_ref, h1tab_ref, selfex_ref, den_ref, w2_ref, b1_ref,
             as2_ref, ad2_ref, h2_ref, as2c_ref, ad2c_ref, ec2c_ref,
             se2c_ref):
    selfex = selfex_ref[:, 0:8]
    den = den_ref[0, :, 0:8] + den_ref[1, :, 0:8] + selfex
    rden = 1.0 / (den + 1e-16)
    cols = []
    for hd in range(HEADS):
        num = scat1_ref[hd] + selfex[:, hd:hd + 1] * h1tab_ref[hd]
        cols.append(num * rden[:, hd:hd + 1])
    x1 = jnp.concatenate(cols, axis=1) + b1_ref[...]
    x1 = jnp.maximum(x1, 0.0)
    h2 = jnp.dot(x1, w2_ref[...], preferred_element_type=_f32)
    a_s2 = (h2 * as2_ref[...]).sum(-1, keepdims=True)
    a_d2 = (h2 * ad2_ref[...]).sum(-1, keepdims=True)
    ec2 = jnp.exp(-_leaky(CAP + a_d2))
    selfex2 = jnp.exp(_leaky(a_s2 + a_d2)) * ec2
    h2_ref[...] = h2
    as2c_ref[...] = a_s2
    ad2c_ref[...] = a_d2
    ec2c_ref[...] = ec2
    se2c_ref[...] = selfex2


def _t6_call(scat1, h1tab, selfex1, den1p, W2, b1, as2, ad2):
    grid = N // BM
    return pl.pallas_call(
        _t6_body,
        grid=(grid,),
        in_specs=[
            pl.BlockSpec((HEADS, BM, HID), lambda i: (0, i, 0)),
            pl.BlockSpec((HEADS, BM, HID), lambda i: (0, i, 0)),
            pl.BlockSpec((BM, 16), lambda i: (i, 0)),
            pl.BlockSpec((2, BM, 16), lambda i: (0, i, 0)),
            pl.BlockSpec((HEADS * HID, OUT_DIM), lambda i: (0, 0)),
            pl.BlockSpec((1, HEADS * HID), lambda i: (0, 0)),
            pl.BlockSpec((1, OUT_DIM), lambda i: (0, 0)),
            pl.BlockSpec((1, OUT_DIM), lambda i: (0, 0)),
        ],
        out_specs=[
            pl.BlockSpec((BM, OUT_DIM), lambda i: (i, 0)),
            pl.BlockSpec((BM, 1), lambda i: (i, 0)),
            pl.BlockSpec((BM, 1), lambda i: (i, 0)),
            pl.BlockSpec((BM, 1), lambda i: (i, 0)),
            pl.BlockSpec((BM, 1), lambda i: (i, 0)),
        ],
        out_shape=[
            jax.ShapeDtypeStruct((N, OUT_DIM), _f32),
            jax.ShapeDtypeStruct((N, 1), _f32),
            jax.ShapeDtypeStruct((N, 1), _f32),
            jax.ShapeDtypeStruct((N, 1), _f32),
            jax.ShapeDtypeStruct((N, 1), _f32),
        ],
    )(scat1, h1tab, selfex1, den1p, W2, b1.reshape(1, -1), as2, ad2)


def _t7_body(scat2_ref, den2_ref, se2_ref, h2_ref, b2_ref, o_ref):
    selfex2 = se2_ref[...]
    den = den2_ref[0, :, 0:1] + den2_ref[1, :, 0:1] + selfex2
    num = scat2_ref[0] + scat2_ref[1] + selfex2 * h2_ref[...]
    out2 = num / (den + 1e-16) + b2_ref[...]
    part = out2.sum(axis=0, keepdims=True) * (1.0 / N)

    @pl.when(pl.program_id(0) == 0)
    def _():
        o_ref[...] = jnp.zeros_like(o_ref)

    o_ref[...] += part


def _t7_call(scat2, den2p, se2, h2, b2):
    grid = N // BM
    return pl.pallas_call(
        _t7_body,
        grid=(grid,),
        in_specs=[
            pl.BlockSpec((2, BM, OUT_DIM), lambda i: (0, i, 0)),
            pl.BlockSpec((2, BM, 16), lambda i: (0, i, 0)),
            pl.BlockSpec((BM, 1), lambda i: (i, 0)),
            pl.BlockSpec((BM, OUT_DIM), lambda i: (i, 0)),
            pl.BlockSpec((1, OUT_DIM), lambda i: (0, 0)),
        ],
        out_specs=pl.BlockSpec((1, OUT_DIM), lambda i: (0, 0)),
        out_shape=jax.ShapeDtypeStruct((1, OUT_DIM), _f32),
    )(scat2, den2p, se2, h2, b2.reshape(1, -1))


def kernel(x, edge_index, edge_attr, We, be, W1, as1, ad1, b1, W2, as2, ad2,
           b2):
    esrc = edge_index[0]
    edst = edge_index[1]
    s32 = _s1_call(edst, edge_attr)
    h1tab, tabs1, tabd1, tabec1, selfex1 = _t1_call(x, s32, We, be, W1, as1,
                                                    ad1)
    ex1, den1p = _s2_call(esrc, edst, tabs1, tabd1, tabec1)
    h1flat = h1tab.reshape(HEADS * N, HID)
    scat1 = _s4_call(esrc, edst, ex1, h1flat)
    h2, as2c, ad2c, ec2c, se2c = _t6_call(scat1, h1tab, selfex1, den1p, W2,
                                          b1, as2, ad2)
    den2p = _s5a_call(esrc, edst, as2c.reshape(N), ad2c.reshape(N),
                      ec2c.reshape(N))
    scat2 = _s5b_call(esrc, edst, as2c.reshape(N), ad2c.reshape(N),
                      ec2c.reshape(N), h2)
    out = _t7_call(scat2, den2p, se2c, h2, b2)
    return out[0]
